# baseline (speedup 1.0000x reference)
import jax
import jax.numpy as jnp
from jax import lax
from jax.experimental import pallas as pl
from jax.experimental.pallas import tpu as pltpu
from jax.experimental.pallas import tpu_sc as plsc

_B, _S, _D = 4, 8192, 2048
_G = 64
_NG = _D // _G
_QMAX = 127.0

_ROWS = 1024
_SPLIT = 8

_NC, _NS, _L = 1, 16, 16
_NW = _NC * _NS
_CPW = _D // _NW


def _scale_gather_sc(inv_perm_i32, group_scales):
    mesh = plsc.VectorSubcoreMesh(core_axis_name="c", subcore_axis_name="s",
                                  num_cores=_NC)

    def body(inv_hbm, gs_hbm, out_hbm, inv_v, g_v, sv_v, sem):
        wid = lax.axis_index("s") * _NC + lax.axis_index("c")
        base = wid * _CPW
        pltpu.sync_copy(inv_hbm.at[pl.ds(base, _CPW)], inv_v)
        for i in range(_CPW // _L):
            g_v[pl.ds(i * _L, _L)] = lax.shift_right_logical(
                inv_v[pl.ds(i * _L, _L)], 6)
        pltpu.async_copy(gs_hbm.at[g_v], sv_v, sem).wait()
        pltpu.sync_copy(sv_v, out_hbm.at[pl.ds(base, _CPW)])

    return pl.kernel(
        body,
        mesh=mesh,
        out_type=jax.ShapeDtypeStruct((_D,), jnp.float32),
        scratch_types=[
            pltpu.VMEM((_CPW,), jnp.int32),
            pltpu.VMEM((_CPW,), jnp.int32),
            pltpu.VMEM((_CPW,), jnp.float32),
            pltpu.SemaphoreType.DMA,
        ],
    )(inv_perm_i32, group_scales)


def _quant_inline_body(inv_ref, gs_ref, x_ref, o_ref):
    g = (inv_ref[...] // _G).astype(jnp.int32)
    ids = lax.broadcasted_iota(jnp.int32, (_NG, _D), 0)
    onehot = (g == ids)
    s = jnp.sum(jnp.where(onehot, gs_ref[...], 0.0), axis=0, keepdims=True)
    xv = x_ref[...]
    q = jnp.clip(jnp.round(xv / s), -_QMAX, _QMAX)
    o_ref[...] = q * s


def _quant_svec_body(s_ref, x_ref, y_in_ref, o_ref):
    del y_in_ref
    s = s_ref[...]
    xv = x_ref[...]
    q = jnp.clip(jnp.round(xv / s), -_QMAX, _QMAX)
    o_ref[...] = q * s


def kernel(x, perm, inv_perm, group_scales):
    del perm
    inv_i32 = inv_perm.astype(jnp.int32)
    gs_f32 = group_scales.astype(jnp.float32)
    s_vec = _scale_gather_sc(inv_i32, gs_f32)

    xf = x.reshape(_B * _S, _D)
    n_steps = xf.shape[0] // _ROWS

    y_half = pl.pallas_call(
        _quant_inline_body,
        grid=(_SPLIT,),
        in_specs=[
            pl.BlockSpec((1, _D), lambda i: (0, 0)),
            pl.BlockSpec((_NG, 1), lambda i: (0, 0)),
            pl.BlockSpec((_ROWS, _D), lambda i: (i, 0)),
        ],
        out_specs=pl.BlockSpec((_ROWS, _D), lambda i: (i, 0)),
        out_shape=jax.ShapeDtypeStruct(xf.shape, x.dtype),
    )(inv_i32.reshape(1, _D), gs_f32.reshape(_NG, 1), xf)

    out = pl.pallas_call(
        _quant_svec_body,
        grid=(n_steps - _SPLIT,),
        in_specs=[
            pl.BlockSpec((1, _D), lambda i: (0, 0)),
            pl.BlockSpec((_ROWS, _D), lambda i: (i + _SPLIT, 0)),
            pl.BlockSpec(memory_space=pltpu.MemorySpace.HBM),
        ],
        out_specs=pl.BlockSpec((_ROWS, _D), lambda i: (i + _SPLIT, 0)),
        out_shape=jax.ShapeDtypeStruct(xf.shape, x.dtype),
        input_output_aliases={2: 0},
    )(s_vec.reshape(1, _D), xf, y_half)
    return out.reshape(x.shape)


# speedup vs baseline: 1.0568x; 1.0568x over previous
import jax
import jax.numpy as jnp
from jax import lax
from jax.experimental import pallas as pl
from jax.experimental.pallas import tpu as pltpu
from jax.experimental.pallas import tpu_sc as plsc

_B, _S, _D = 4, 8192, 2048
_G = 64
_NG = _D // _G
_QMAX = 127.0

_ROWS = 1024
_SPLIT = 8

_NC, _NS, _L = 1, 16, 16
_NW = _NC * _NS
_CPW = _D // _NW


def _scale_gather_sc(inv_perm_i32, group_scales):
    mesh = plsc.VectorSubcoreMesh(core_axis_name="c", subcore_axis_name="s",
                                  num_cores=_NC)

    def body(inv_hbm, gs_hbm, out_hbm, inv_v, g_v, sv_v, sem):
        wid = lax.axis_index("s") * _NC + lax.axis_index("c")
        base = wid * _CPW
        pltpu.sync_copy(inv_hbm.at[pl.ds(base, _CPW)], inv_v)
        for i in range(_CPW // _L):
            g_v[pl.ds(i * _L, _L)] = lax.shift_right_logical(
                inv_v[pl.ds(i * _L, _L)], 6)
        pltpu.async_copy(gs_hbm.at[g_v], sv_v, sem).wait()
        pltpu.sync_copy(sv_v, out_hbm.at[pl.ds(base, _CPW)])

    return pl.kernel(
        body,
        mesh=mesh,
        out_type=jax.ShapeDtypeStruct((_D,), jnp.float32),
        scratch_types=[
            pltpu.VMEM((_CPW,), jnp.int32),
            pltpu.VMEM((_CPW,), jnp.int32),
            pltpu.VMEM((_CPW,), jnp.float32),
            pltpu.SemaphoreType.DMA,
        ],
    )(inv_perm_i32, group_scales)


def _scale_gather_scs(inv_perm_i32, group_scales):
    mesh = plsc.ScalarSubcoreMesh(axis_name="c", num_cores=1)
    chunk = 512

    def body(inv_hbm, gs_hbm, out_hbm, inv_s, gs_s, sv_s):
        pltpu.sync_copy(gs_hbm, gs_s)
        for c in range(_D // chunk):
            pltpu.sync_copy(inv_hbm.at[pl.ds(c * chunk, chunk)], inv_s)

            def lookup(i, carry):
                g = lax.shift_right_logical(inv_s[i], 6)
                sv_s[i] = gs_s[g]
                return carry

            lax.fori_loop(0, chunk, lookup, 0)
            pltpu.sync_copy(sv_s, out_hbm.at[pl.ds(c * chunk, chunk)])

    return pl.kernel(
        body,
        mesh=mesh,
        out_type=jax.ShapeDtypeStruct((_D,), jnp.float32),
        scratch_types=[
            pltpu.SMEM((chunk,), jnp.int32),
            pltpu.SMEM((_NG,), jnp.float32),
            pltpu.SMEM((chunk,), jnp.float32),
        ],
    )(inv_perm_i32, group_scales)


def _quant_inline_body(inv_ref, gs_ref, x_ref, o_ref):
    g = (inv_ref[...] // _G).astype(jnp.int32)
    ids = lax.broadcasted_iota(jnp.int32, (_NG, _D), 0)
    onehot = (g == ids)
    s = jnp.sum(jnp.where(onehot, gs_ref[...], 0.0), axis=0, keepdims=True)
    xv = x_ref[...]
    q = jnp.clip(jnp.round(xv / s), -_QMAX, _QMAX)
    o_ref[...] = q * s


def _quant_svec_body(s_ref, x_ref, y_in_ref, o_ref):
    del y_in_ref
    s = s_ref[...]
    xv = x_ref[...]
    q = jnp.clip(jnp.round(xv / s), -_QMAX, _QMAX)
    o_ref[...] = q * s


def kernel(x, perm, inv_perm, group_scales):
    del perm
    inv_i32 = inv_perm.astype(jnp.int32)
    gs_f32 = group_scales.astype(jnp.float32)
    s_vec = _scale_gather_scs(inv_i32, gs_f32)

    xf = x.reshape(_B * _S, _D)
    n_steps = xf.shape[0] // _ROWS

    y_half = pl.pallas_call(
        _quant_inline_body,
        grid=(_SPLIT,),
        in_specs=[
            pl.BlockSpec((1, _D), lambda i: (0, 0)),
            pl.BlockSpec((_NG, 1), lambda i: (0, 0)),
            pl.BlockSpec((_ROWS, _D), lambda i: (i, 0)),
        ],
        out_specs=pl.BlockSpec((_ROWS, _D), lambda i: (i, 0)),
        out_shape=jax.ShapeDtypeStruct(xf.shape, x.dtype),
    )(inv_i32.reshape(1, _D), gs_f32.reshape(_NG, 1), xf)

    out = pl.pallas_call(
        _quant_svec_body,
        grid=(n_steps - _SPLIT,),
        in_specs=[
            pl.BlockSpec((1, _D), lambda i: (0, 0)),
            pl.BlockSpec((_ROWS, _D), lambda i: (i + _SPLIT, 0)),
            pl.BlockSpec(memory_space=pltpu.MemorySpace.HBM),
        ],
        out_specs=pl.BlockSpec((_ROWS, _D), lambda i: (i + _SPLIT, 0)),
        out_shape=jax.ShapeDtypeStruct(xf.shape, x.dtype),
        input_output_aliases={2: 0},
    )(s_vec.reshape(1, _D), xf, y_half)
    return out.reshape(x.shape)


# speedup vs baseline: 1.0569x; 1.0002x over previous
import jax
import jax.numpy as jnp
from jax import lax
from jax.experimental import pallas as pl
from jax.experimental.pallas import tpu as pltpu
from jax.experimental.pallas import tpu_sc as plsc

_B, _S, _D = 4, 8192, 2048
_G = 64
_NG = _D // _G
_QMAX = 127.0

_ROWS = 1024
_SPLIT = 8

def _scale_gather_scs(inv_perm_i32, group_scales):
    mesh = plsc.ScalarSubcoreMesh(axis_name="c", num_cores=1)
    chunk = 512

    def body(inv_hbm, gs_hbm, out_hbm, inv_s, gs_s, sv_s):
        pltpu.sync_copy(gs_hbm, gs_s)
        for c in range(_D // chunk):
            pltpu.sync_copy(inv_hbm.at[pl.ds(c * chunk, chunk)], inv_s)

            def lookup(i, carry):
                g = lax.shift_right_logical(inv_s[i], 6)
                sv_s[i] = gs_s[g]
                return carry

            lax.fori_loop(0, chunk, lookup, 0)
            pltpu.sync_copy(sv_s, out_hbm.at[pl.ds(c * chunk, chunk)])

    return pl.kernel(
        body,
        mesh=mesh,
        out_type=jax.ShapeDtypeStruct((_D,), jnp.float32),
        scratch_types=[
            pltpu.SMEM((chunk,), jnp.int32),
            pltpu.SMEM((_NG,), jnp.float32),
            pltpu.SMEM((chunk,), jnp.float32),
        ],
    )(inv_perm_i32, group_scales)


def _quant_inline_body(inv_ref, gs_ref, x_ref, o_ref):
    g = (inv_ref[...] // _G).astype(jnp.int32)
    ids = lax.broadcasted_iota(jnp.int32, (_NG, _D), 0)
    onehot = (g == ids)
    s = jnp.sum(jnp.where(onehot, gs_ref[...], 0.0), axis=0, keepdims=True)
    xv = x_ref[...]
    q = jnp.clip(jnp.round(xv / s), -_QMAX, _QMAX)
    o_ref[...] = q * s


def _quant_svec_body(s_ref, x_ref, y_in_ref, o_ref):
    del y_in_ref
    s = s_ref[...]
    xv = x_ref[...]
    q = jnp.clip(jnp.round(xv / s), -_QMAX, _QMAX)
    o_ref[...] = q * s


def kernel(x, perm, inv_perm, group_scales):
    del perm
    inv_i32 = inv_perm.astype(jnp.int32)
    gs_f32 = group_scales.astype(jnp.float32)
    s_vec = _scale_gather_scs(inv_i32, gs_f32)

    xf = x.reshape(_B * _S, _D)
    n_steps = xf.shape[0] // _ROWS

    y_half = pl.pallas_call(
        _quant_inline_body,
        grid=(_SPLIT,),
        in_specs=[
            pl.BlockSpec((1, _D), lambda i: (0, 0)),
            pl.BlockSpec((_NG, 1), lambda i: (0, 0)),
            pl.BlockSpec((_ROWS, _D), lambda i: (i, 0)),
        ],
        out_specs=pl.BlockSpec((_ROWS, _D), lambda i: (i, 0)),
        out_shape=jax.ShapeDtypeStruct(xf.shape, x.dtype),
    )(inv_i32.reshape(1, _D), gs_f32.reshape(_NG, 1), xf)

    out = pl.pallas_call(
        _quant_svec_body,
        grid=(n_steps - _SPLIT,),
        in_specs=[
            pl.BlockSpec((1, _D), lambda i: (0, 0)),
            pl.BlockSpec((_ROWS, _D), lambda i: (i + _SPLIT, 0)),
            pl.BlockSpec(memory_space=pltpu.MemorySpace.HBM),
        ],
        out_specs=pl.BlockSpec((_ROWS, _D), lambda i: (i + _SPLIT, 0)),
        out_shape=jax.ShapeDtypeStruct(xf.shape, x.dtype),
        input_output_aliases={2: 0},
    )(s_vec.reshape(1, _D), xf, y_half)
    return out.reshape(x.shape)


# speedup vs baseline: 1.0575x; 1.0006x over previous
import jax
import jax.numpy as jnp
from jax import lax
from jax.experimental import pallas as pl
from jax.experimental.pallas import tpu as pltpu
from jax.experimental.pallas import tpu_sc as plsc

_B, _S, _D = 4, 8192, 2048
_G = 64
_NG = _D // _G
_QMAX = 127.0

_ROWS = 1024
_SPLIT = 8

def _scale_gather_scs(inv_perm_i32, group_scales):
    mesh = plsc.ScalarSubcoreMesh(axis_name="c", num_cores=1)
    chunk = 1024

    def body(inv_hbm, gs_hbm, out_hbm, inv_s, gs_s, sv_s):
        pltpu.sync_copy(gs_hbm, gs_s)
        for c in range(_D // chunk):
            pltpu.sync_copy(inv_hbm.at[pl.ds(c * chunk, chunk)], inv_s)

            def lookup(i, carry):
                g = lax.shift_right_logical(inv_s[i], 6)
                sv_s[i] = gs_s[g]
                return carry

            lax.fori_loop(0, chunk, lookup, 0)
            pltpu.sync_copy(sv_s, out_hbm.at[pl.ds(c * chunk, chunk)])

    return pl.kernel(
        body,
        mesh=mesh,
        out_type=jax.ShapeDtypeStruct((_D,), jnp.float32),
        scratch_types=[
            pltpu.SMEM((chunk,), jnp.int32),
            pltpu.SMEM((_NG,), jnp.float32),
            pltpu.SMEM((chunk,), jnp.float32),
        ],
    )(inv_perm_i32, group_scales)


def _quant_inline_body(inv_ref, gs_ref, x_ref, o_ref):
    g = (inv_ref[...] // _G).astype(jnp.int32)
    ids = lax.broadcasted_iota(jnp.int32, (_NG, _D), 0)
    onehot = (g == ids)
    s = jnp.sum(jnp.where(onehot, gs_ref[...], 0.0), axis=0, keepdims=True)
    xv = x_ref[...]
    q = jnp.clip(jnp.round(xv / s), -_QMAX, _QMAX)
    o_ref[...] = q * s


def _quant_svec_body(s_ref, x_ref, y_in_ref, o_ref):
    del y_in_ref
    s = s_ref[...]
    xv = x_ref[...]
    q = jnp.clip(jnp.round(xv / s), -_QMAX, _QMAX)
    o_ref[...] = q * s


def kernel(x, perm, inv_perm, group_scales):
    del perm
    inv_i32 = inv_perm.astype(jnp.int32)
    gs_f32 = group_scales.astype(jnp.float32)
    s_vec = _scale_gather_scs(inv_i32, gs_f32)

    xf = x.reshape(_B * _S, _D)
    n_steps = xf.shape[0] // _ROWS

    y_half = pl.pallas_call(
        _quant_inline_body,
        grid=(_SPLIT,),
        in_specs=[
            pl.BlockSpec((1, _D), lambda i: (0, 0)),
            pl.BlockSpec((_NG, 1), lambda i: (0, 0)),
            pl.BlockSpec((_ROWS, _D), lambda i: (i, 0)),
        ],
        out_specs=pl.BlockSpec((_ROWS, _D), lambda i: (i, 0)),
        out_shape=jax.ShapeDtypeStruct(xf.shape, x.dtype),
    )(inv_i32.reshape(1, _D), gs_f32.reshape(_NG, 1), xf)

    out = pl.pallas_call(
        _quant_svec_body,
        grid=(n_steps - _SPLIT,),
        in_specs=[
            pl.BlockSpec((1, _D), lambda i: (0, 0)),
            pl.BlockSpec((_ROWS, _D), lambda i: (i + _SPLIT, 0)),
            pl.BlockSpec(memory_space=pltpu.MemorySpace.HBM),
        ],
        out_specs=pl.BlockSpec((_ROWS, _D), lambda i: (i + _SPLIT, 0)),
        out_shape=jax.ShapeDtypeStruct(xf.shape, x.dtype),
        input_output_aliases={2: 0},
    )(s_vec.reshape(1, _D), xf, y_half)
    return out.reshape(x.shape)
